# Initial kernel scaffold; baseline (speedup 1.0000x reference)
import jax
import jax.numpy as jnp
from jax.experimental import pallas as pl


def kernel(x, edge_index, edge_weight, params):
    raise NotImplementedError("write your pallas kernel here")


# speedup vs baseline: 3.4390x; 3.4390x over previous
import functools

import jax
import jax.numpy as jnp
from jax import lax
from jax.experimental import pallas as pl
from jax.experimental.pallas import tpu as pltpu
from jax.experimental.pallas import tpu_sc as plsc

N = 10000
E = 320000
D = 128

NC = 2
NS = 16
NW = NC * NS
CHUNK = 128
CPT = -(-E // (NW * CHUNK))
EPT = CPT * CHUNK
E_PAD = EPT * NW
ROWS_PT = 624
ROWS_TAIL = N - ROWS_PT * NS

_mesh = plsc.VectorSubcoreMesh(core_axis_name="c", subcore_axis_name="s")


@functools.partial(
    pl.kernel,
    out_type=jax.ShapeDtypeStruct((NC, N, D), jnp.float32),
    mesh=_mesh,
    scratch_types=[
        pltpu.VMEM_SHARED((N, D), jnp.float32),
        pltpu.VMEM((CHUNK,), jnp.int32),
        pltpu.VMEM((CHUNK,), jnp.int32),
        pltpu.VMEM((CHUNK,), jnp.float32),
        pltpu.VMEM((CHUNK, D), jnp.float32),
        pltpu.SemaphoreType.DMA,
    ],
)
def _sc_msgpass(h_hbm, src_hbm, dst_hbm, w_hbm, zero_hbm, out_hbm,
                agg_sh, src_v, dst_v, w_v, rows_v, sem):
    c = lax.axis_index("c")
    s = lax.axis_index("s")
    wid = c * NS + s

    pltpu.sync_copy(zero_hbm.at[pl.ds(0, ROWS_PT)],
                    agg_sh.at[pl.ds(s * ROWS_PT, ROWS_PT)])

    @pl.when(s == NS - 1)
    def _zero_tail():
        pltpu.sync_copy(zero_hbm.at[pl.ds(0, ROWS_TAIL)],
                        agg_sh.at[pl.ds(NS * ROWS_PT, ROWS_TAIL)])

    plsc.subcore_barrier()

    base = wid * EPT

    def chunk_body(g, carry):
        e0 = base + g * CHUNK
        pltpu.sync_copy(src_hbm.at[pl.ds(e0, CHUNK)], src_v)
        pltpu.sync_copy(dst_hbm.at[pl.ds(e0, CHUNK)], dst_v)
        pltpu.sync_copy(w_hbm.at[pl.ds(e0, CHUNK)], w_v)
        pltpu.async_copy(h_hbm.at[src_v], rows_v, sem).wait()

        def row_body(q, carry2):
            wblk = w_v[pl.ds(q * 16, 16)]
            for r2 in range(16):
                r = q * 16 + r2
                w16 = jnp.broadcast_to(wblk[r2], (16,))
                for cc in range(D // 16):
                    sl = pl.ds(cc * 16, 16)
                    rows_v[r, sl] = rows_v[r, sl] * w16
            return carry2

        lax.fori_loop(0, CHUNK // 16, row_body, 0)
        pltpu.sync_copy(rows_v, agg_sh.at[dst_v], add=True)
        return carry

    lax.fori_loop(0, CPT, chunk_body, 0)
    plsc.subcore_barrier()

    sl = pl.ds(s * ROWS_PT, ROWS_PT)
    pltpu.sync_copy(agg_sh.at[sl], out_hbm.at[c, sl])

    @pl.when(s == NS - 1)
    def _out_tail():
        sl2 = pl.ds(NS * ROWS_PT, ROWS_TAIL)
        pltpu.sync_copy(agg_sh.at[sl2], out_hbm.at[c, sl2])


RB = 2000


def _mm_body(a_ref, w_ref, b_ref, o_ref, *, relu):
    acc = jnp.dot(a_ref[0] + a_ref[1], w_ref[...],
                  preferred_element_type=jnp.float32) + b_ref[...]
    if relu:
        acc = jnp.maximum(acc, 0.0)
    o_ref[...] = acc


def _mm_res_body(a_ref, w_ref, b_ref, res_ref, o_ref):
    acc = jnp.dot(a_ref[0] + a_ref[1], w_ref[...],
                  preferred_element_type=jnp.float32) + b_ref[...]
    acc = jnp.maximum(acc, 0.0)
    o_ref[...] = (res_ref[...] + acc) * 0.5


def _tc_matmul(agg, W, b, relu):
    return pl.pallas_call(
        functools.partial(_mm_body, relu=relu),
        grid=(N // RB,),
        in_specs=[
            pl.BlockSpec((NC, RB, D), lambda i: (0, i, 0)),
            pl.BlockSpec((D, D), lambda i: (0, 0)),
            pl.BlockSpec((1, D), lambda i: (0, 0)),
        ],
        out_specs=pl.BlockSpec((RB, D), lambda i: (i, 0)),
        out_shape=jax.ShapeDtypeStruct((N, D), jnp.float32),
    )(agg, W, b.reshape(1, D))


def _tc_matmul_res(agg, W, b, res):
    return pl.pallas_call(
        _mm_res_body,
        grid=(N // RB,),
        in_specs=[
            pl.BlockSpec((NC, RB, D), lambda i: (0, i, 0)),
            pl.BlockSpec((D, D), lambda i: (0, 0)),
            pl.BlockSpec((1, D), lambda i: (0, 0)),
            pl.BlockSpec((RB, D), lambda i: (i, 0)),
        ],
        out_specs=pl.BlockSpec((RB, D), lambda i: (i, 0)),
        out_shape=jax.ShapeDtypeStruct((N, D), jnp.float32),
    )(agg, W, b.reshape(1, D), res)


def kernel(x, edge_index, edge_weight, params):
    src = edge_index[0]
    dst = edge_index[1]
    pad = E_PAD - E
    src_p = jnp.pad(src, (0, pad))
    dst_p = jnp.pad(dst, (0, pad))
    w_p = jnp.pad(edge_weight, (0, pad))
    zero = jnp.zeros((ROWS_PT, D), jnp.float32)

    def S(h):
        return _sc_msgpass(h, src_p, dst_p, w_p, zero)

    p_in = params["conv_in"]
    h = _tc_matmul(S(x), p_in["W"], p_in["b"], relu=True)
    for bp in params["blocks"]:
        h1 = _tc_matmul(S(h), bp["conv1"]["W"], bp["conv1"]["b"], relu=True)
        h = _tc_matmul_res(S(h1), bp["conv2"]["W"], bp["conv2"]["b"], h)
    p_out = params["conv_out"]
    x_out = _tc_matmul(S(h), p_out["W"], p_out["b"], relu=False)
    return (x_out, h)
